# baseline (speedup 1.0000x reference)
import functools

import jax
import jax.numpy as jnp
from jax import lax
from jax.experimental import pallas as pl
from jax.experimental.pallas import tpu as pltpu
from jax.experimental.pallas import tpu_sc as plsc

V = 1000012
D = 16
F = 26
FP = 32
B = 16384
BF2 = B * FP

VB = 16384
VZ = ((V + VB - 1) // VB) * VB
VF = VZ

NC = 2
NS = 16
NW = NC * NS
PER_W = BF2 // NW
CHUNK = 2048
N_CHUNKS = PER_W // CHUNK


def _lin_body(t_ref, out_ref):
    i = pl.program_id(0)
    e = t_ref[...]
    col = lax.broadcasted_iota(jnp.int32, (D, VB), 1) + i * VB
    e = jnp.where(col < V, e, 0.0)
    g = e.T.reshape(VB // 8, 8, D)
    for j in range(8):
        out_ref[:, D * j:D * (j + 1)] = g[:, j, :]


def _linearize(emb_t):
    return pl.pallas_call(
        _lin_body,
        grid=(VZ // VB,),
        in_specs=[pl.BlockSpec((D, VB), lambda i: (0, i))],
        out_specs=pl.BlockSpec((VB // 8, 128), lambda i: (i, 0)),
        out_shape=jax.ShapeDtypeStruct((VZ // 8, 128), jnp.float32),
    )(emb_t)


def _sc_gather_body(emb_hbm, fc_hbm, idx_hbm, out_emb, out_fc,
                    idx_v, rows_v, fcrows_v, sem_e, sem_f):
    wid = lax.axis_index("s") * NC + lax.axis_index("c")
    base = wid * PER_W
    for g in range(N_CHUNKS):
        off = base + g * CHUNK
        pltpu.sync_copy(idx_hbm.at[pl.ds(off, CHUNK)], idx_v)
        cp_e = pltpu.async_copy(emb_hbm.at[idx_v], rows_v, sem_e)
        cp_f = pltpu.async_copy(fc_hbm.at[idx_v], fcrows_v, sem_f)
        cp_e.wait()
        cp_f.wait()
        pltpu.sync_copy(rows_v, out_emb.at[pl.ds(off, CHUNK)])
        pltpu.sync_copy(fcrows_v, out_fc.at[pl.ds(off, CHUNK)])


def _sc_gather(emb_lin, fc_lin, idx):
    mesh = plsc.VectorSubcoreMesh(core_axis_name="c", subcore_axis_name="s")
    return pl.kernel(
        _sc_gather_body,
        out_type=(
            jax.ShapeDtypeStruct((BF2, D), jnp.float32),
            jax.ShapeDtypeStruct((BF2,), jnp.float32),
        ),
        mesh=mesh,
        scratch_types=[
            pltpu.VMEM((CHUNK,), jnp.int32),
            pltpu.VMEM((CHUNK, D), jnp.float32),
            pltpu.VMEM((CHUNK,), jnp.float32),
            pltpu.SemaphoreType.DMA,
            pltpu.SemaphoreType.DMA,
        ],
        compiler_params=pltpu.CompilerParams(use_tc_tiling_on_sc=False),
    )(emb_lin, fc_lin, idx)


BLK = 1024
NK = FP * D // 128


def _tc_body(e_ref, fcg_ref, scal_ref, w0_ref,
             b0_ref, w1_ref, b1_ref, w2_ref, b2_ref, w3_ref, out_ref):
    w0 = w0_ref[...]
    sums = jnp.zeros((BLK, D), jnp.float32)
    sqs = jnp.zeros((BLK, D), jnp.float32)
    h = jnp.zeros((BLK, 256), jnp.float32)
    for k in range(NK):
        ek = e_ref[k]
        r = lax.broadcasted_iota(jnp.int32, (128, D), 0) + 128 * k
        c = lax.broadcasted_iota(jnp.int32, (128, D), 1)
        smk = jnp.where(((r % D) == c) & (r < F * D), 1.0, 0.0)
        sums = sums + jnp.dot(ek, smk, preferred_element_type=jnp.float32)
        sqs = sqs + jnp.dot(ek * ek, smk, preferred_element_type=jnp.float32)
        h = h + jnp.dot(ek, w0[128 * k:128 * (k + 1), :],
                        preferred_element_type=jnp.float32)
    inter = 0.5 * jnp.sum(sums * sums - sqs, axis=1, keepdims=True)
    fc_sum = jnp.sum(fcg_ref[...], axis=1, keepdims=True)
    h = jnp.maximum(h + b0_ref[...], 0.0)
    h = jnp.maximum(jnp.dot(h, w1_ref[...], preferred_element_type=jnp.float32)
                    + b1_ref[...], 0.0)
    h = jnp.maximum(jnp.dot(h, w2_ref[...], preferred_element_type=jnp.float32)
                    + b2_ref[...], 0.0)
    mlp = jnp.sum(h * w3_ref[...], axis=1, keepdims=True)
    z = inter + fc_sum + mlp + scal_ref[0]
    out_ref[...] = 1.0 / (1.0 + jnp.exp(-z))


def _tc_head(e_ks, fc_mat, scal, w0p, b0, w1, b1, w2, b2, w3t):
    grid = (B // BLK,)
    return pl.pallas_call(
        _tc_body,
        grid=grid,
        in_specs=[
            pl.BlockSpec((NK, BLK, 128), lambda i: (0, i, 0)),
            pl.BlockSpec((BLK, FP), lambda i: (i, 0)),
            pl.BlockSpec(memory_space=pltpu.SMEM),
            pl.BlockSpec((FP * D, 256), lambda i: (0, 0)),
            pl.BlockSpec((1, 256), lambda i: (0, 0)),
            pl.BlockSpec((256, 128), lambda i: (0, 0)),
            pl.BlockSpec((1, 128), lambda i: (0, 0)),
            pl.BlockSpec((128, 64), lambda i: (0, 0)),
            pl.BlockSpec((1, 64), lambda i: (0, 0)),
            pl.BlockSpec((1, 64), lambda i: (0, 0)),
        ],
        out_specs=pl.BlockSpec((BLK, 1), lambda i: (i, 0)),
        out_shape=jax.ShapeDtypeStruct((B, 1), jnp.float32),
    )(e_ks, fc_mat, scal, w0p, b0, w1, b1, w2, b2, w3t)


def kernel(x, emb_table, fc_table, bias, W0, b0, W1, b1, W2, b2, W3, b3):
    pad_idx = V + jax.lax.iota(jnp.int32, B * (FP - F)).reshape(B, FP - F) \
        % (VZ - V - (FP - F))
    xi = x.astype(jnp.int32)
    xp = jnp.concatenate([xi, pad_idx], axis=1)
    idx = xp.reshape(B, NK, 8).transpose(1, 0, 2).reshape(BF2)
    fc_pad = jnp.pad(fc_table, ((0, VF - V), (0, 0))).reshape(VF)
    emb_lin2 = _linearize(emb_table.T)
    e_flat, fc_flat = _sc_gather(emb_lin2.reshape(VZ, D), fc_pad, idx)
    e_ks = e_flat.reshape(NK, B, 128)
    fc_mat = fc_flat.reshape(NK, B, 8).transpose(1, 0, 2).reshape(B, FP)
    scal = (bias + b3).astype(jnp.float32)
    w0p = jnp.pad(W0, ((0, (FP - F) * D), (0, 0)))
    out = _tc_head(e_ks, fc_mat, scal,
                   w0p, b0.reshape(1, 256), W1, b1.reshape(1, 128),
                   W2, b2.reshape(1, 64), W3.reshape(1, 64))
    return out.reshape(B)


# speedup vs baseline: 1.0871x; 1.0871x over previous
import functools

import jax
import jax.numpy as jnp
from jax import lax
from jax.experimental import pallas as pl
from jax.experimental.pallas import tpu as pltpu
from jax.experimental.pallas import tpu_sc as plsc

V = 1000012
D = 16
F = 26
FP = 32
B = 16384
BF2 = B * FP

VB = 16384
VZ = ((V + VB - 1) // VB) * VB
VF = VZ

NC = 2
NS = 16
NW = NC * NS
PER_W = BF2 // NW
CHUNK = 2048
N_CHUNKS = PER_W // CHUNK


def _lin_body(t_ref, out_ref):
    i = pl.program_id(0)
    e = t_ref[...]
    col = lax.broadcasted_iota(jnp.int32, (D, VB), 1) + i * VB
    e = jnp.where(col < V, e, 0.0)
    g = e.T.reshape(VB // 8, 8, D)
    for j in range(8):
        out_ref[:, D * j:D * (j + 1)] = g[:, j, :]


def _linearize(emb_t):
    return pl.pallas_call(
        _lin_body,
        grid=(VZ // VB,),
        in_specs=[pl.BlockSpec((D, VB), lambda i: (0, i))],
        out_specs=pl.BlockSpec((VB // 8, 128), lambda i: (i, 0)),
        out_shape=jax.ShapeDtypeStruct((VZ // 8, 128), jnp.float32),
    )(emb_t)


def _sc_gather_body(emb_hbm, fc_hbm, idx_hbm, out_emb, out_fc,
                    idx_v, rows_v, fcrows_v, sem_e, sem_f):
    wid = lax.axis_index("s") * NC + lax.axis_index("c")
    base = wid * PER_W
    for g in range(N_CHUNKS):
        off = base + g * CHUNK
        pltpu.sync_copy(idx_hbm.at[pl.ds(off, CHUNK)], idx_v)
        cp_e = pltpu.async_copy(emb_hbm.at[idx_v], rows_v, sem_e)
        cp_f = pltpu.async_copy(fc_hbm.at[idx_v], fcrows_v, sem_f)
        cp_e.wait()
        cp_f.wait()
        pltpu.sync_copy(rows_v, out_emb.at[pl.ds(off, CHUNK)])
        pltpu.sync_copy(fcrows_v, out_fc.at[pl.ds(off, CHUNK)])


def _sc_gather(emb_lin, fc_lin, idx):
    mesh = plsc.VectorSubcoreMesh(core_axis_name="c", subcore_axis_name="s")
    return pl.kernel(
        _sc_gather_body,
        out_type=(
            jax.ShapeDtypeStruct((BF2, D), jnp.float32),
            jax.ShapeDtypeStruct((BF2,), jnp.float32),
        ),
        mesh=mesh,
        scratch_types=[
            pltpu.VMEM((CHUNK,), jnp.int32),
            pltpu.VMEM((CHUNK, D), jnp.float32),
            pltpu.VMEM((CHUNK,), jnp.float32),
            pltpu.SemaphoreType.DMA,
            pltpu.SemaphoreType.DMA,
        ],
        compiler_params=pltpu.CompilerParams(use_tc_tiling_on_sc=False),
    )(emb_lin, fc_lin, idx)


BLK = 1024
NK = FP * D // 128


def _tc_body(e_ref, fcg_ref, scal_ref, w0_ref,
             b0_ref, w1_ref, b1_ref, w2_ref, b2_ref, w3_ref, out_ref):
    e4 = e_ref[...].reshape(BLK, NK, 128)
    w0 = w0_ref[...]
    sums = jnp.zeros((BLK, D), jnp.float32)
    sqs = jnp.zeros((BLK, D), jnp.float32)
    h = jnp.zeros((BLK, 256), jnp.float32)
    for k in range(NK):
        ek = e4[:, k, :]
        r = lax.broadcasted_iota(jnp.int32, (128, D), 0) + 128 * k
        c = lax.broadcasted_iota(jnp.int32, (128, D), 1)
        smk = jnp.where(((r % D) == c) & (r < F * D), 1.0, 0.0)
        sums = sums + jnp.dot(ek, smk, preferred_element_type=jnp.float32)
        sqs = sqs + jnp.dot(ek * ek, smk, preferred_element_type=jnp.float32)
        h = h + jnp.dot(ek, w0[128 * k:128 * (k + 1), :],
                        preferred_element_type=jnp.float32)
    inter = 0.5 * jnp.sum(sums * sums - sqs, axis=1, keepdims=True)
    fc_sum = jnp.sum(fcg_ref[...], axis=1, keepdims=True)
    h = jnp.maximum(h + b0_ref[...], 0.0)
    h = jnp.maximum(jnp.dot(h, w1_ref[...], preferred_element_type=jnp.float32)
                    + b1_ref[...], 0.0)
    h = jnp.maximum(jnp.dot(h, w2_ref[...], preferred_element_type=jnp.float32)
                    + b2_ref[...], 0.0)
    mlp = jnp.sum(h * w3_ref[...], axis=1, keepdims=True)
    z = inter + fc_sum + mlp + scal_ref[0]
    out_ref[...] = 1.0 / (1.0 + jnp.exp(-z))


def _tc_head(e_ks, fc_mat, scal, w0p, b0, w1, b1, w2, b2, w3t):
    grid = (B // BLK,)
    return pl.pallas_call(
        _tc_body,
        grid=grid,
        in_specs=[
            pl.BlockSpec((BLK * NK, 128), lambda i: (i, 0)),
            pl.BlockSpec((BLK, FP), lambda i: (i, 0)),
            pl.BlockSpec(memory_space=pltpu.SMEM),
            pl.BlockSpec((FP * D, 256), lambda i: (0, 0)),
            pl.BlockSpec((1, 256), lambda i: (0, 0)),
            pl.BlockSpec((256, 128), lambda i: (0, 0)),
            pl.BlockSpec((1, 128), lambda i: (0, 0)),
            pl.BlockSpec((128, 64), lambda i: (0, 0)),
            pl.BlockSpec((1, 64), lambda i: (0, 0)),
            pl.BlockSpec((1, 64), lambda i: (0, 0)),
        ],
        out_specs=pl.BlockSpec((BLK, 1), lambda i: (i, 0)),
        out_shape=jax.ShapeDtypeStruct((B, 1), jnp.float32),
    )(e_ks, fc_mat, scal, w0p, b0, w1, b1, w2, b2, w3t)


def kernel(x, emb_table, fc_table, bias, W0, b0, W1, b1, W2, b2, W3, b3):
    pad_idx = V + jax.lax.iota(jnp.int32, B * (FP - F)).reshape(B, FP - F) \
        % (VZ - V - (FP - F))
    xi = x.astype(jnp.int32)
    idx = jnp.concatenate([xi, pad_idx], axis=1).reshape(BF2)
    fc_pad = jnp.pad(fc_table, ((0, VF - V), (0, 0))).reshape(VF)
    emb_lin2 = _linearize(emb_table.T)
    e_flat, fc_flat = _sc_gather(emb_lin2.reshape(VZ, D), fc_pad, idx)
    e_ks = e_flat.reshape(BF2 * D // 128, 128)
    fc_mat = fc_flat.reshape(B, FP)
    scal = (bias + b3).astype(jnp.float32)
    w0p = jnp.pad(W0, ((0, (FP - F) * D), (0, 0)))
    out = _tc_head(e_ks, fc_mat, scal,
                   w0p, b0.reshape(1, 256), W1, b1.reshape(1, 128),
                   W2, b2.reshape(1, 64), W3.reshape(1, 64))
    return out.reshape(B)


# speedup vs baseline: 1.0971x; 1.0092x over previous
import functools

import jax
import jax.numpy as jnp
from jax import lax
from jax.experimental import pallas as pl
from jax.experimental.pallas import tpu as pltpu
from jax.experimental.pallas import tpu_sc as plsc

V = 1000012
D = 16
F = 26
FP = 32
B = 16384
BF2 = B * FP

VB = 16384
VZ = ((V + VB - 1) // VB) * VB
VF = VZ

NC = 2
NS = 16
NW = NC * NS
PER_W = BF2 // NW
CHUNK = 2048
N_CHUNKS = PER_W // CHUNK


def _lin_body(t_ref, out_ref):
    i = pl.program_id(0)
    e = t_ref[...]
    col = lax.broadcasted_iota(jnp.int32, (D, VB), 1) + i * VB
    e = jnp.where(col < V, e, 0.0)
    g = e.T.reshape(VB // 8, 8, D)
    for j in range(8):
        out_ref[:, D * j:D * (j + 1)] = g[:, j, :]


def _linearize(emb_t):
    return pl.pallas_call(
        _lin_body,
        grid=(VZ // VB,),
        in_specs=[pl.BlockSpec((D, VB), lambda i: (0, i))],
        out_specs=pl.BlockSpec((VB // 8, 128), lambda i: (i, 0)),
        out_shape=jax.ShapeDtypeStruct((VZ // 8, 128), jnp.float32),
    )(emb_t)


def _sc_gather_body(emb_hbm, fc_hbm, idx_hbm, out_emb, out_fc,
                    idx_v, rows_v, fcrows_v, sem_e, sem_f):
    wid = lax.axis_index("s") * NC + lax.axis_index("c")
    base = wid * PER_W

    def start(g):
        buf = g % 2
        pltpu.sync_copy(idx_hbm.at[pl.ds(base + g * CHUNK, CHUNK)],
                        idx_v.at[buf])
        he = pltpu.async_copy(emb_hbm.at[idx_v.at[buf]], rows_v.at[buf], sem_e)
        hf = pltpu.async_copy(fc_hbm.at[idx_v.at[buf]], fcrows_v.at[buf], sem_f)
        return he, hf

    he, hf = start(0)
    for g in range(N_CHUNKS):
        buf = g % 2
        he.wait()
        hf.wait()
        if g + 1 < N_CHUNKS:
            he, hf = start(g + 1)
        off = base + g * CHUNK
        pltpu.sync_copy(rows_v.at[buf], out_emb.at[pl.ds(off, CHUNK)])
        pltpu.sync_copy(fcrows_v.at[buf], out_fc.at[pl.ds(off, CHUNK)])


def _sc_gather(emb_lin, fc_lin, idx):
    mesh = plsc.VectorSubcoreMesh(core_axis_name="c", subcore_axis_name="s")
    return pl.kernel(
        _sc_gather_body,
        out_type=(
            jax.ShapeDtypeStruct((BF2, D), jnp.float32),
            jax.ShapeDtypeStruct((BF2,), jnp.float32),
        ),
        mesh=mesh,
        scratch_types=[
            pltpu.VMEM((2, CHUNK), jnp.int32),
            pltpu.VMEM((2, CHUNK, D), jnp.float32),
            pltpu.VMEM((2, CHUNK), jnp.float32),
            pltpu.SemaphoreType.DMA,
            pltpu.SemaphoreType.DMA,
        ],
        compiler_params=pltpu.CompilerParams(use_tc_tiling_on_sc=False),
    )(emb_lin, fc_lin, idx)


BLK = 1024
NK = FP * D // 128


def _tc_body(e_ref, fcg_ref, scal_ref, w0_ref,
             b0_ref, w1_ref, b1_ref, w2_ref, b2_ref, w3_ref, out_ref):
    e4 = e_ref[...].reshape(BLK, NK, 128)
    w0 = w0_ref[...]
    sums = jnp.zeros((BLK, D), jnp.float32)
    sqs = jnp.zeros((BLK, D), jnp.float32)
    h = jnp.zeros((BLK, 256), jnp.float32)
    for k in range(NK):
        ek = e4[:, k, :]
        r = lax.broadcasted_iota(jnp.int32, (128, D), 0) + 128 * k
        c = lax.broadcasted_iota(jnp.int32, (128, D), 1)
        smk = jnp.where(((r % D) == c) & (r < F * D), 1.0, 0.0)
        sums = sums + jnp.dot(ek, smk, preferred_element_type=jnp.float32)
        sqs = sqs + jnp.dot(ek * ek, smk, preferred_element_type=jnp.float32)
        h = h + jnp.dot(ek, w0[128 * k:128 * (k + 1), :],
                        preferred_element_type=jnp.float32)
    inter = 0.5 * jnp.sum(sums * sums - sqs, axis=1, keepdims=True)
    fc_sum = jnp.sum(fcg_ref[...], axis=1, keepdims=True)
    h = jnp.maximum(h + b0_ref[...], 0.0)
    h = jnp.maximum(jnp.dot(h, w1_ref[...], preferred_element_type=jnp.float32)
                    + b1_ref[...], 0.0)
    h = jnp.maximum(jnp.dot(h, w2_ref[...], preferred_element_type=jnp.float32)
                    + b2_ref[...], 0.0)
    mlp = jnp.sum(h * w3_ref[...], axis=1, keepdims=True)
    z = inter + fc_sum + mlp + scal_ref[0]
    out_ref[...] = 1.0 / (1.0 + jnp.exp(-z))


def _tc_head(e_ks, fc_mat, scal, w0p, b0, w1, b1, w2, b2, w3t):
    grid = (B // BLK,)
    return pl.pallas_call(
        _tc_body,
        grid=grid,
        in_specs=[
            pl.BlockSpec((BLK * NK, 128), lambda i: (i, 0)),
            pl.BlockSpec((BLK, FP), lambda i: (i, 0)),
            pl.BlockSpec(memory_space=pltpu.SMEM),
            pl.BlockSpec((FP * D, 256), lambda i: (0, 0)),
            pl.BlockSpec((1, 256), lambda i: (0, 0)),
            pl.BlockSpec((256, 128), lambda i: (0, 0)),
            pl.BlockSpec((1, 128), lambda i: (0, 0)),
            pl.BlockSpec((128, 64), lambda i: (0, 0)),
            pl.BlockSpec((1, 64), lambda i: (0, 0)),
            pl.BlockSpec((1, 64), lambda i: (0, 0)),
        ],
        out_specs=pl.BlockSpec((BLK, 1), lambda i: (i, 0)),
        out_shape=jax.ShapeDtypeStruct((B, 1), jnp.float32),
    )(e_ks, fc_mat, scal, w0p, b0, w1, b1, w2, b2, w3t)


def kernel(x, emb_table, fc_table, bias, W0, b0, W1, b1, W2, b2, W3, b3):
    pad_idx = V + jax.lax.iota(jnp.int32, B * (FP - F)).reshape(B, FP - F) \
        % (VZ - V - (FP - F))
    xi = x.astype(jnp.int32)
    idx = jnp.concatenate([xi, pad_idx], axis=1).reshape(BF2)
    fc_pad = jnp.pad(fc_table, ((0, VF - V), (0, 0))).reshape(VF)
    emb_lin2 = _linearize(emb_table.T)
    e_flat, fc_flat = _sc_gather(emb_lin2.reshape(VZ, D), fc_pad, idx)
    e_ks = e_flat.reshape(BF2 * D // 128, 128)
    fc_mat = fc_flat.reshape(B, FP)
    scal = (bias + b3).astype(jnp.float32)
    w0p = jnp.pad(W0, ((0, (FP - F) * D), (0, 0)))
    out = _tc_head(e_ks, fc_mat, scal,
                   w0p, b0.reshape(1, 256), W1, b1.reshape(1, 128),
                   W2, b2.reshape(1, 64), W3.reshape(1, 64))
    return out.reshape(B)
